# baseline (speedup 1.0000x reference)
import jax
import jax.numpy as jnp
from jax.experimental import pallas as pl
from jax.experimental.pallas import tpu as pltpu

MEM_ROWS = 4096
VOCAB = 32000
EMBED = 128
HOPS = 3
M_TILE = 64


def _mc_kernel(x_ref, a_ref, c_ref, o_ref):
    w = jnp.concatenate([a_ref[...], c_ref[...]], axis=0)
    o_ref[...] = jax.lax.dot_general(
        x_ref[...], w, (((1,), (1,)), ((), ())),
        preferred_element_type=jnp.float32)


def _hops_kernel(mc_ref, q_ref, b_ref, u_ref):
    u = jax.lax.dot_general(
        q_ref[...], b_ref[...], (((1,), (1,)), ((), ())),
        preferred_element_type=jnp.float32)
    mc = mc_ref[...]
    m = mc[:, :EMBED]
    c = mc[:, EMBED:]
    for _ in range(HOPS):
        logits = jax.lax.dot_general(
            u, m, (((1,), (1,)), ((), ())),
            preferred_element_type=jnp.float32)
        logits = logits - jnp.max(logits, axis=-1, keepdims=True)
        p = jnp.exp(logits)
        p = p / jnp.sum(p, axis=-1, keepdims=True)
        o = jnp.dot(p, c, preferred_element_type=jnp.float32)
        u = u + o
    u_ref[...] = u


def kernel(memory, query, A, B, C):
    x = memory.reshape(MEM_ROWS, VOCAB)

    mc = pl.pallas_call(
        _mc_kernel,
        grid=(MEM_ROWS // M_TILE,),
        in_specs=[
            pl.BlockSpec((M_TILE, VOCAB), lambda i: (i, 0)),
            pl.BlockSpec((EMBED, VOCAB), lambda i: (0, 0)),
            pl.BlockSpec((EMBED, VOCAB), lambda i: (0, 0)),
        ],
        out_specs=pl.BlockSpec((M_TILE, 2 * EMBED), lambda i: (i, 0)),
        out_shape=jax.ShapeDtypeStruct((MEM_ROWS, 2 * EMBED), jnp.float32),
        compiler_params=pltpu.CompilerParams(
            dimension_semantics=("parallel",),
            vmem_limit_bytes=60 * 1024 * 1024,
        ),
    )(x, A, C)

    u = pl.pallas_call(
        _hops_kernel,
        out_shape=jax.ShapeDtypeStruct((1, EMBED), jnp.float32),
        compiler_params=pltpu.CompilerParams(
            vmem_limit_bytes=40 * 1024 * 1024,
        ),
    )(mc, query, B)
    return u


# speedup vs baseline: 1.8430x; 1.8430x over previous
import jax
import jax.numpy as jnp
from jax.experimental import pallas as pl
from jax.experimental.pallas import tpu as pltpu

MEM_ROWS = 4096
VOCAB = 32000
EMBED = 128
HOPS = 3
M_TILE = 2048
V_TILE = 1280


def _mc_kernel(x_ref, a_ref, c_ref, o_ref):
    j = pl.program_id(1)
    w = jnp.concatenate([a_ref[...], c_ref[...]], axis=0)
    part = jax.lax.dot_general(
        x_ref[...].astype(jnp.bfloat16), w.astype(jnp.bfloat16),
        (((1,), (1,)), ((), ())),
        preferred_element_type=jnp.float32)

    @pl.when(j == 0)
    def _():
        o_ref[...] = part

    @pl.when(j > 0)
    def _():
        o_ref[...] = o_ref[...] + part


def _hops_kernel(mc_ref, q_ref, b_ref, u_ref):
    u = jax.lax.dot_general(
        q_ref[...], b_ref[...], (((1,), (1,)), ((), ())),
        preferred_element_type=jnp.float32)
    mc = mc_ref[...]
    m = mc[:, :EMBED]
    c = mc[:, EMBED:]
    for _ in range(HOPS):
        logits = jax.lax.dot_general(
            u, m, (((1,), (1,)), ((), ())),
            preferred_element_type=jnp.float32)
        logits = logits - jnp.max(logits, axis=-1, keepdims=True)
        p = jnp.exp(logits)
        p = p / jnp.sum(p, axis=-1, keepdims=True)
        o = jnp.dot(p, c, preferred_element_type=jnp.float32)
        u = u + o
    u_ref[...] = u


def kernel(memory, query, A, B, C):
    x = memory.reshape(MEM_ROWS, VOCAB)

    mc = pl.pallas_call(
        _mc_kernel,
        grid=(MEM_ROWS // M_TILE, VOCAB // V_TILE),
        in_specs=[
            pl.BlockSpec((M_TILE, V_TILE), lambda i, j: (i, j)),
            pl.BlockSpec((EMBED, V_TILE), lambda i, j: (0, j)),
            pl.BlockSpec((EMBED, V_TILE), lambda i, j: (0, j)),
        ],
        out_specs=pl.BlockSpec((M_TILE, 2 * EMBED), lambda i, j: (i, 0)),
        out_shape=jax.ShapeDtypeStruct((MEM_ROWS, 2 * EMBED), jnp.float32),
        compiler_params=pltpu.CompilerParams(
            dimension_semantics=("parallel", "arbitrary"),
            vmem_limit_bytes=60 * 1024 * 1024,
        ),
    )(x, A, C)

    u = pl.pallas_call(
        _hops_kernel,
        out_shape=jax.ShapeDtypeStruct((1, EMBED), jnp.float32),
        compiler_params=pltpu.CompilerParams(
            vmem_limit_bytes=40 * 1024 * 1024,
        ),
    )(mc, query, B)
    return u


# speedup vs baseline: 1.9245x; 1.0442x over previous
import jax
import jax.numpy as jnp
from jax.experimental import pallas as pl
from jax.experimental.pallas import tpu as pltpu

MEM_ROWS = 4096
VOCAB = 32000
EMBED = 128
HOPS = 3
V_TILE = 640
J_STEPS = VOCAB // V_TILE // 2


def _mc_kernel(x_ref, a_ref, c_ref, q_ref, b_ref, mcp_ref, u0p_ref,
               acc_ref, u0_acc_ref):
    j = pl.program_id(1)
    w = jnp.concatenate([a_ref[...], c_ref[...]], axis=0)
    part = jax.lax.dot_general(
        x_ref[...].astype(jnp.bfloat16), w.astype(jnp.bfloat16),
        (((1,), (1,)), ((), ())),
        preferred_element_type=jnp.float32)
    u0part = jax.lax.dot_general(
        q_ref[...].astype(jnp.bfloat16), b_ref[...].astype(jnp.bfloat16),
        (((1,), (1,)), ((), ())),
        preferred_element_type=jnp.float32)

    @pl.when(j == 0)
    def _():
        acc_ref[...] = part
        u0_acc_ref[...] = u0part

    @pl.when(j > 0)
    def _():
        acc_ref[...] = acc_ref[...] + part
        u0_acc_ref[...] = u0_acc_ref[...] + u0part

    @pl.when(j == J_STEPS - 1)
    def _():
        mcp_ref[0] = acc_ref[...]
        u0p_ref[0] = u0_acc_ref[...]


def _hops_kernel(mcp_ref, u0p_ref, u_ref):
    mc = mcp_ref[0] + mcp_ref[1]
    u = u0p_ref[0] + u0p_ref[1]
    m = mc[:, :EMBED]
    c = mc[:, EMBED:]
    for _ in range(HOPS):
        logits = jax.lax.dot_general(
            u, m, (((1,), (1,)), ((), ())),
            preferred_element_type=jnp.float32)
        logits = logits - jnp.max(logits, axis=-1, keepdims=True)
        p = jnp.exp(logits)
        p = p / jnp.sum(p, axis=-1, keepdims=True)
        o = jnp.dot(p, c, preferred_element_type=jnp.float32)
        u = u + o
    u_ref[...] = u


def kernel(memory, query, A, B, C):
    x = memory.reshape(MEM_ROWS, VOCAB)

    mcp, u0p = pl.pallas_call(
        _mc_kernel,
        grid=(2, J_STEPS),
        in_specs=[
            pl.BlockSpec((MEM_ROWS, V_TILE), lambda kv, j: (0, kv * J_STEPS + j)),
            pl.BlockSpec((EMBED, V_TILE), lambda kv, j: (0, kv * J_STEPS + j)),
            pl.BlockSpec((EMBED, V_TILE), lambda kv, j: (0, kv * J_STEPS + j)),
            pl.BlockSpec((1, V_TILE), lambda kv, j: (0, kv * J_STEPS + j)),
            pl.BlockSpec((EMBED, V_TILE), lambda kv, j: (0, kv * J_STEPS + j)),
        ],
        out_specs=[
            pl.BlockSpec((1, MEM_ROWS, 2 * EMBED), lambda kv, j: (kv, 0, 0)),
            pl.BlockSpec((1, 1, EMBED), lambda kv, j: (kv, 0, 0)),
        ],
        out_shape=[
            jax.ShapeDtypeStruct((2, MEM_ROWS, 2 * EMBED), jnp.float32),
            jax.ShapeDtypeStruct((2, 1, EMBED), jnp.float32),
        ],
        scratch_shapes=[
            pltpu.VMEM((MEM_ROWS, 2 * EMBED), jnp.float32),
            pltpu.VMEM((1, EMBED), jnp.float32),
        ],
        compiler_params=pltpu.CompilerParams(
            dimension_semantics=("parallel", "arbitrary"),
            vmem_limit_bytes=60 * 1024 * 1024,
        ),
    )(x, A, C, query, B)

    u = pl.pallas_call(
        _hops_kernel,
        out_shape=jax.ShapeDtypeStruct((1, EMBED), jnp.float32),
        compiler_params=pltpu.CompilerParams(
            vmem_limit_bytes=40 * 1024 * 1024,
        ),
    )(mcp, u0p)
    return u
